# baseline (speedup 1.0000x reference)
import functools

import jax
import jax.numpy as jnp
from jax.experimental import pallas as pl
from jax.experimental.pallas import tpu as pltpu

_W = 8192


def _loss_kernel(x_ref, t_ref, o_ref, sum_acc, tgt_acc, *, nsteps, width, ncols):
    j = pl.program_id(0)
    x = x_ref[...]
    b, w = x.shape
    e = jnp.exp(x)
    iota = jax.lax.broadcasted_iota(jnp.int32, (b, w), 1)
    is_t = iota == (t_ref[...] - j * width)
    te = jnp.sum(jnp.where(is_t, e, 0.0), axis=1, keepdims=True)

    @pl.when(j == 0)
    def _init():
        sum_acc[...] = jnp.sum(e, axis=1, keepdims=True)
        tgt_acc[...] = te

    @pl.when((j > 0) & (j < nsteps - 1))
    def _accum():
        sum_acc[...] += jnp.sum(e, axis=1, keepdims=True)
        tgt_acc[...] += te

    @pl.when(j == nsteps - 1)
    def _finish():
        ev = jnp.where(iota < ncols - j * width, e, 0.0)
        s = sum_acc[...] + jnp.sum(ev, axis=1, keepdims=True)
        p = (tgt_acc[...] + te) / s
        o_ref[...] = -jnp.mean(jnp.log(p + 1e-6)).reshape(1, 1)


def kernel(input, target):
    b, c = input.shape
    nsteps = pl.cdiv(c, _W)
    t2 = target.astype(jnp.int32).reshape(b, 1)
    out = pl.pallas_call(
        functools.partial(_loss_kernel, nsteps=nsteps, width=_W, ncols=c),
        grid=(nsteps,),
        in_specs=[
            pl.BlockSpec((b, _W), lambda j: (0, j)),
            pl.BlockSpec((b, 1), lambda j: (0, 0)),
        ],
        out_specs=pl.BlockSpec((1, 1), lambda j: (0, 0)),
        out_shape=jax.ShapeDtypeStruct((1, 1), jnp.float32),
        scratch_shapes=[
            pltpu.VMEM((b, 1), jnp.float32),
            pltpu.VMEM((b, 1), jnp.float32),
        ],
    )(input, t2)
    return out[0, 0]
